# baseline (speedup 1.0000x reference)
import functools
import math

import jax
import jax.numpy as jnp
from jax.experimental import pallas as pl
from jax.experimental.pallas import tpu as pltpu

_BLK = 2000


def _gelu(v):
    return 0.5 * v * (1.0 + jax.lax.erf(v * (1.0 / math.sqrt(2.0))))


def _body(batch_ref, x_ref, u_ref,
          nw1_ref, nb1_ref, nw2_ref, nb2_ref,
          qw1_ref, qb1_ref, qw2_ref, qb2_ref,
          out_ref, uq_s, m_s, d_s, acc_s, *, nblocks, nseg):
    i = pl.program_id(0)

    @pl.when(i == 0)
    def _init():
        uqh = _gelu(jnp.dot(u_ref[:], qw1_ref[:],
                            preferred_element_type=jnp.float32) + qb1_ref[:])
        uq_s[:] = jnp.dot(uqh, qw2_ref[:],
                          preferred_element_type=jnp.float32) + qb2_ref[:]
        m_s[:] = jnp.full(m_s.shape, -1e30, jnp.float32)
        d_s[:] = jnp.zeros(d_s.shape, jnp.float32)
        acc_s[:] = jnp.zeros(acc_s.shape, jnp.float32)

    x = x_ref[:]
    h = _gelu(jnp.dot(x, nw1_ref[:], preferred_element_type=jnp.float32)
              + nb1_ref[:])
    xn = jnp.dot(h.astype(jnp.bfloat16), nw2_ref[:],
                 preferred_element_type=jnp.float32) + nb2_ref[:]

    c = xn.shape[1]
    xn_b = xn.astype(jnp.bfloat16)
    gate_all = jax.lax.dot_general(
        xn_b, uq_s[:].astype(jnp.bfloat16), (((1,), (1,)), ((), ())),
        preferred_element_type=jnp.float32) * (1.0 / math.sqrt(c))
    seg = batch_ref[0, 0, :]
    onehot = seg[:, None] == jax.lax.broadcasted_iota(
        jnp.int32, (1, nseg), 1)
    gate_own = jnp.where(onehot, gate_all, -jnp.inf)

    m_old = m_s[0, :]
    m_new = jnp.maximum(m_old, jnp.max(gate_own, axis=0))
    scale = jnp.exp(m_old - m_new)
    p = jnp.where(onehot, jnp.exp(gate_all - m_new[None, :]), 0.0)

    d_s[0, :] = d_s[0, :] * scale + jnp.sum(p, axis=0)
    acc_s[:] = acc_s[:] * scale[:, None] + jax.lax.dot_general(
        p.astype(jnp.bfloat16), xn_b, (((0,), (0,)), ((), ())),
        preferred_element_type=jnp.float32)
    m_s[0, :] = m_new

    @pl.when(i == nblocks - 1)
    def _fin():
        out_ref[:] = acc_s[:] / (d_s[0, :][:, None] + 1e-16)


def kernel(x, u, batch, size, node_w1, node_b1, node_w2, node_b2,
           ques_w1, ques_b1, ques_w2, ques_b2):
    n, d = x.shape
    nseg, c = u.shape
    nblocks = n // _BLK
    assert nblocks * _BLK == n

    batch3 = batch.reshape(nblocks, 1, _BLK)
    x = x.astype(jnp.bfloat16)
    node_w1 = node_w1.astype(jnp.bfloat16)
    node_w2 = node_w2.astype(jnp.bfloat16)
    nb1 = node_b1.reshape(1, c)
    nb2 = node_b2.reshape(1, c)
    qb1 = ques_b1.reshape(1, c)
    qb2 = ques_b2.reshape(1, c)

    full = lambda shape: pl.BlockSpec(shape, lambda i: (0,) * len(shape))
    out = pl.pallas_call(
        functools.partial(_body, nblocks=nblocks, nseg=nseg),
        grid=(nblocks,),
        in_specs=[
            pl.BlockSpec((1, 1, _BLK), lambda i: (i, 0, 0)),
            pl.BlockSpec((_BLK, d), lambda i: (i, 0)),
            full((nseg, c)),
            full((d, c)), full((1, c)), full((c, c)), full((1, c)),
            full((c, c)), full((1, c)), full((c, c)), full((1, c)),
        ],
        out_specs=pl.BlockSpec((nseg, c), lambda i: (0, 0)),
        out_shape=jax.ShapeDtypeStruct((nseg, c), jnp.float32),
        scratch_shapes=[
            pltpu.VMEM((nseg, c), jnp.float32),
            pltpu.VMEM((1, nseg), jnp.float32),
            pltpu.VMEM((1, nseg), jnp.float32),
            pltpu.VMEM((nseg, c), jnp.float32),
        ],
        compiler_params=pltpu.CompilerParams(
            dimension_semantics=("arbitrary",)),
    )(batch3, x, u, node_w1, nb1, node_w2, nb2, ques_w1, qb1, ques_w2, qb2)

    return out + jnp.zeros((), out.dtype) * jnp.asarray(size, out.dtype)


# speedup vs baseline: 1.5877x; 1.5877x over previous
import functools
import math

import jax
import jax.numpy as jnp
from jax.experimental import pallas as pl
from jax.experimental.pallas import tpu as pltpu

_BLK = 2000


def _gelu(v):
    return 0.5 * v * (1.0 + jax.lax.erf(v * (1.0 / math.sqrt(2.0))))


def _body(batch_ref, x_ref, u_ref,
          nw1_ref, nb1_ref, nw2_ref, nb2_ref,
          qw1_ref, qb1_ref, qw2_ref, qb2_ref,
          out_ref, uq_s, m_s, d_s, acc_s, *, nblocks, nseg):
    i = pl.program_id(0)

    @pl.when(i == 0)
    def _init():
        uqh = _gelu(jnp.dot(u_ref[:], qw1_ref[:],
                            preferred_element_type=jnp.float32) + qb1_ref[:])
        uq = jnp.dot(uqh, qw2_ref[:],
                     preferred_element_type=jnp.float32) + qb2_ref[:]
        uq_s[:] = uq * (1.0 / math.sqrt(uq.shape[1]))
        m_s[:] = jnp.full(m_s.shape, -1e30, jnp.float32)
        d_s[:] = jnp.zeros(d_s.shape, jnp.float32)
        acc_s[:] = jnp.zeros(acc_s.shape, jnp.float32)

    x = x_ref[:]
    h = _gelu(jnp.dot(x, nw1_ref[:], preferred_element_type=jnp.float32)
              + nb1_ref[:])
    xn = jnp.dot(h, nw2_ref[:], preferred_element_type=jnp.float32) + nb2_ref[:]

    gate_t = jax.lax.dot_general(
        uq_s[:], xn, (((1,), (1,)), ((), ())),
        preferred_element_type=jnp.float32)
    seg = batch_ref[0]
    onehot = jax.lax.broadcasted_iota(jnp.int32, (nseg, 1), 0) == seg
    gate_own = jnp.where(onehot, gate_t, -jnp.inf)

    m_old = m_s[:]
    m_new = jnp.maximum(m_old, jnp.max(gate_own, axis=1, keepdims=True))
    scale = jnp.exp(m_old - m_new)
    p = jnp.where(onehot, jnp.exp(gate_t - m_new), 0.0)

    d_s[:] = d_s[:] * scale + jnp.sum(p, axis=1, keepdims=True)
    acc_s[:] = acc_s[:] * scale + jax.lax.dot_general(
        p, xn, (((1,), (0,)), ((), ())),
        preferred_element_type=jnp.float32)
    m_s[:] = m_new

    @pl.when(i == nblocks - 1)
    def _fin():
        out_ref[:] = acc_s[:] / (d_s[:] + 1e-16)


def kernel(x, u, batch, size, node_w1, node_b1, node_w2, node_b2,
           ques_w1, ques_b1, ques_w2, ques_b2):
    n, d = x.shape
    nseg, c = u.shape
    nblocks = n // _BLK
    assert nblocks * _BLK == n

    batch3 = batch.reshape(nblocks, 1, _BLK)
    nb1 = node_b1.reshape(1, c)
    nb2 = node_b2.reshape(1, c)
    qb1 = ques_b1.reshape(1, c)
    qb2 = ques_b2.reshape(1, c)

    full = lambda shape: pl.BlockSpec(shape, lambda i: (0,) * len(shape))
    out = pl.pallas_call(
        functools.partial(_body, nblocks=nblocks, nseg=nseg),
        grid=(nblocks,),
        in_specs=[
            pl.BlockSpec((1, 1, _BLK), lambda i: (i, 0, 0)),
            pl.BlockSpec((_BLK, d), lambda i: (i, 0)),
            full((nseg, c)),
            full((d, c)), full((1, c)), full((c, c)), full((1, c)),
            full((c, c)), full((1, c)), full((c, c)), full((1, c)),
        ],
        out_specs=pl.BlockSpec((nseg, c), lambda i: (0, 0)),
        out_shape=jax.ShapeDtypeStruct((nseg, c), jnp.float32),
        scratch_shapes=[
            pltpu.VMEM((nseg, c), jnp.float32),
            pltpu.VMEM((nseg, 1), jnp.float32),
            pltpu.VMEM((nseg, 1), jnp.float32),
            pltpu.VMEM((nseg, c), jnp.float32),
        ],
        compiler_params=pltpu.CompilerParams(
            dimension_semantics=("arbitrary",)),
    )(batch3, x, u, node_w1, nb1, node_w2, nb2, ques_w1, qb1, ques_w2, qb2)

    return out + jnp.zeros((), out.dtype) * jnp.asarray(size, out.dtype)


# speedup vs baseline: 1.7767x; 1.1190x over previous
import functools
import math

import jax
import jax.numpy as jnp
from jax.experimental import pallas as pl
from jax.experimental.pallas import tpu as pltpu

_BLK = 4000
_SPLIT = 2


def _gelu(v):
    return 0.5 * v * (1.0 + jax.lax.erf(v * (1.0 / math.sqrt(2.0))))


def _body(batch_ref, x_ref, u_ref,
          nw1_ref, nb1_ref, nw2_ref, nb2_ref,
          qw1_ref, qb1_ref, qw2_ref, qb2_ref,
          out_ref, uq_s, m_s, d_s, acc_s, *, nblocks, nseg):
    i = pl.program_id(0)

    @pl.when(i == 0)
    def _init():
        uqh = _gelu(jnp.dot(u_ref[:], qw1_ref[:],
                            preferred_element_type=jnp.float32) + qb1_ref[:])
        uq = jnp.dot(uqh, qw2_ref[:],
                     preferred_element_type=jnp.float32) + qb2_ref[:]
        uq_s[:] = uq * (1.0 / math.sqrt(uq.shape[1]))
        m_s[:] = jnp.full(m_s.shape, -1e30, jnp.float32)
        d_s[:] = jnp.zeros(d_s.shape, jnp.float32)
        acc_s[:] = jnp.zeros(acc_s.shape, jnp.float32)

    sub = _BLK // _SPLIT
    iota_col = jax.lax.broadcasted_iota(jnp.int32, (nseg, 1), 0)
    xns, onehots, gates, bmaxs = [], [], [], []
    for k in range(_SPLIT):
        x = x_ref[pl.ds(k * sub, sub), :]
        h = _gelu(jnp.dot(x, nw1_ref[:], preferred_element_type=jnp.float32)
                  + nb1_ref[:])
        xn = jnp.dot(h, nw2_ref[:],
                     preferred_element_type=jnp.float32) + nb2_ref[:]
        gate_t = jax.lax.dot_general(
            uq_s[:], xn, (((1,), (1,)), ((), ())),
            preferred_element_type=jnp.float32)
        seg = batch_ref[0, :, pl.ds(k * sub, sub)]
        onehot = iota_col == seg
        gate_own = jnp.where(onehot, gate_t, -jnp.inf)
        xns.append(xn)
        onehots.append(onehot)
        gates.append(gate_t)
        bmaxs.append(jnp.max(gate_own, axis=1, keepdims=True))

    m_old = m_s[:]
    m_new = m_old
    for bm in bmaxs:
        m_new = jnp.maximum(m_new, bm)
    scale = jnp.exp(m_old - m_new)

    d_upd = d_s[:] * scale
    acc_upd = acc_s[:] * scale
    for k in range(_SPLIT):
        p = jnp.where(onehots[k], jnp.exp(gates[k] - m_new), 0.0)
        d_upd = d_upd + jnp.sum(p, axis=1, keepdims=True)
        acc_upd = acc_upd + jax.lax.dot_general(
            p, xns[k], (((1,), (0,)), ((), ())),
            preferred_element_type=jnp.float32)
    d_s[:] = d_upd
    acc_s[:] = acc_upd
    m_s[:] = m_new

    @pl.when(i == nblocks - 1)
    def _fin():
        out_ref[:] = acc_s[:] / (d_s[:] + 1e-16)


def kernel(x, u, batch, size, node_w1, node_b1, node_w2, node_b2,
           ques_w1, ques_b1, ques_w2, ques_b2):
    n, d = x.shape
    nseg, c = u.shape
    nblocks = n // _BLK
    assert nblocks * _BLK == n

    batch3 = batch.reshape(nblocks, 1, _BLK)
    nb1 = node_b1.reshape(1, c)
    nb2 = node_b2.reshape(1, c)
    qb1 = ques_b1.reshape(1, c)
    qb2 = ques_b2.reshape(1, c)

    full = lambda shape: pl.BlockSpec(shape, lambda i: (0,) * len(shape))
    out = pl.pallas_call(
        functools.partial(_body, nblocks=nblocks, nseg=nseg),
        grid=(nblocks,),
        in_specs=[
            pl.BlockSpec((1, 1, _BLK), lambda i: (i, 0, 0)),
            pl.BlockSpec((_BLK, d), lambda i: (i, 0)),
            full((nseg, c)),
            full((d, c)), full((1, c)), full((c, c)), full((1, c)),
            full((c, c)), full((1, c)), full((c, c)), full((1, c)),
        ],
        out_specs=pl.BlockSpec((nseg, c), lambda i: (0, 0)),
        out_shape=jax.ShapeDtypeStruct((nseg, c), jnp.float32),
        scratch_shapes=[
            pltpu.VMEM((nseg, c), jnp.float32),
            pltpu.VMEM((nseg, 1), jnp.float32),
            pltpu.VMEM((nseg, 1), jnp.float32),
            pltpu.VMEM((nseg, c), jnp.float32),
        ],
        compiler_params=pltpu.CompilerParams(
            dimension_semantics=("arbitrary",)),
    )(batch3, x, u, node_w1, nb1, node_w2, nb2, ques_w1, qb1, ques_w2, qb2)

    return out + jnp.zeros((), out.dtype) * jnp.asarray(size, out.dtype)
